# baseline (speedup 1.0000x reference)
import functools

import jax
import jax.numpy as jnp
from jax import lax
from jax.experimental import pallas as pl
from jax.experimental.pallas import tpu as pltpu
from jax.experimental.pallas import tpu_sc as plsc

N = 10000
D = 128
E = 320000

NC = 2
NS = 16
CH = 128
K = 160
KBI = 32
NB = K // KBI
EPT = K * CH
EP = EPT * NS
NP = 10240
ROWS_PT = NP // NS

_mesh = plsc.VectorSubcoreMesh(core_axis_name="c", subcore_axis_name="s")


@functools.partial(
    pl.kernel,
    out_type=[
        jax.ShapeDtypeStruct((NS, NP), jnp.float32),
        jax.ShapeDtypeStruct((NS, NP), jnp.float32),
    ],
    mesh=_mesh,
    scratch_types=[
        pltpu.VMEM((EPT,), jnp.int32),
        pltpu.VMEM((NP,), jnp.float32),
    ],
    compiler_params=pltpu.CompilerParams(needs_layout_passes=False),
)
def _deg_kernel(dst0, dst1, degp0, degp1, dst_v, deg_v):
    cid = lax.axis_index("c")
    sid = lax.axis_index("s")
    zeros16 = jnp.zeros((16,), jnp.float32)
    ones16 = jnp.ones((16,), jnp.float32)

    @pl.loop(0, NP // 16)
    def _(i):
        deg_v[pl.ds(i * 16, 16)] = zeros16

    def work(dstb, degpb):
        pltpu.sync_copy(dstb.at[sid], dst_v)

        @pl.loop(0, EPT // 16)
        def _(i):
            idx = dst_v[pl.ds(i * 16, 16)]
            plsc.addupdate_scatter(deg_v, [idx], ones16)

        pltpu.sync_copy(deg_v, degpb.at[sid])

    @pl.when(cid == 0)
    def _():
        work(dst0, degp0)

    @pl.when(cid == 1)
    def _():
        work(dst1, degp1)


@functools.partial(
    pl.kernel,
    out_type=[
        jax.ShapeDtypeStruct((NP, D), jnp.float32),
        jax.ShapeDtypeStruct((NP, D), jnp.float32),
    ],
    mesh=_mesh,
    scratch_types=[
        pltpu.VMEM((KBI, CH), jnp.int32),
        pltpu.VMEM((KBI, CH), jnp.int32),
        pltpu.VMEM((CH, D), jnp.float32),
        pltpu.VMEM((CH, D), jnp.float32),
        pltpu.VMEM_SHARED((NP, D), jnp.float32),
        pltpu.SemaphoreType.DMA,
        pltpu.SemaphoreType.DMA,
    ],
    compiler_params=pltpu.CompilerParams(needs_layout_passes=False),
)
def _prop_kernel(y0, y1, src0, dst0, src1, dst1, zr, z0, z1,
                 src_v, dst_v, rows0_v, rows1_v, agg, sem0, sem1):
    cid = lax.axis_index("c")
    sid = lax.axis_index("s")

    pltpu.sync_copy(zr.at[pl.ds(sid * ROWS_PT, ROWS_PT)],
                    agg.at[pl.ds(sid * ROWS_PT, ROWS_PT)])
    plsc.subcore_barrier()

    def work(yb, srcb, dstb, zb):
        @pl.loop(0, NB)
        def _(i):
            pltpu.sync_copy(srcb.at[sid, pl.ds(i * KBI, KBI)], src_v)
            pltpu.sync_copy(dstb.at[sid, pl.ds(i * KBI, KBI)], dst_v)
            g0 = pltpu.async_copy(yb.at[src_v.at[0]], rows0_v, sem0)
            g1 = pltpu.async_copy(yb.at[src_v.at[1]], rows1_v, sem1)

            @pl.loop(0, KBI, step=2)
            def _(j):
                g0.wait()
                pltpu.sync_copy(rows0_v, agg.at[dst_v.at[j]], add=True)

                @pl.when(j + 2 < KBI)
                def _():
                    pltpu.async_copy(yb.at[src_v.at[j + 2]], rows0_v, sem0)

                g1.wait()
                pltpu.sync_copy(rows1_v, agg.at[dst_v.at[j + 1]], add=True)

                @pl.when(j + 3 < KBI)
                def _():
                    pltpu.async_copy(yb.at[src_v.at[j + 3]], rows1_v, sem1)

    @pl.when(cid == 0)
    def _():
        work(y0, src0, dst0, z0)

    @pl.when(cid == 1)
    def _():
        work(y1, src1, dst1, z1)

    plsc.subcore_barrier()

    @pl.when(cid == 0)
    def _():
        pltpu.sync_copy(agg.at[pl.ds(sid * ROWS_PT, ROWS_PT)],
                        z0.at[pl.ds(sid * ROWS_PT, ROWS_PT)])

    @pl.when(cid == 1)
    def _():
        pltpu.sync_copy(agg.at[pl.ds(sid * ROWS_PT, ROWS_PT)],
                        z1.at[pl.ds(sid * ROWS_PT, ROWS_PT)])


BLK = 512


def _prep_body(degp0_ref, degp1_ref, xp_ref, dinv_ref, y0_ref, y1_ref):
    deg0 = jnp.sum(degp0_ref[...], axis=0) + 1.0
    deg1 = jnp.sum(degp1_ref[...], axis=0) + 1.0
    dinv0 = lax.rsqrt(jnp.maximum(deg0, 1.0))
    dinv1 = lax.rsqrt(jnp.maximum(deg1, 1.0))
    dinv_ref[0, :] = dinv0
    dinv_ref[1, :] = dinv1
    x = xp_ref[...]
    y0_ref[...] = x * dinv0[:, None]
    y1_ref[...] = x * dinv1[:, None]


def _prep(degp0, degp1, xp):
    return pl.pallas_call(
        _prep_body,
        grid=(NP // BLK,),
        in_specs=[
            pl.BlockSpec((NS, BLK), lambda i: (0, i)),
            pl.BlockSpec((NS, BLK), lambda i: (0, i)),
            pl.BlockSpec((BLK, D), lambda i: (i, 0)),
        ],
        out_specs=[
            pl.BlockSpec((2, BLK), lambda i: (0, i)),
            pl.BlockSpec((BLK, D), lambda i: (i, 0)),
            pl.BlockSpec((BLK, D), lambda i: (i, 0)),
        ],
        out_shape=[
            jax.ShapeDtypeStruct((2, NP), jnp.float32),
            jax.ShapeDtypeStruct((NP, D), jnp.float32),
            jax.ShapeDtypeStruct((NP, D), jnp.float32),
        ],
    )(degp0, degp1, xp)


def _mid_body(z0_ref, z1_ref, y0_ref, y1_ref, dinv_ref, w_ref, b_ref,
              o0_ref, o1_ref):
    dinv0 = dinv_ref[0, :]
    dinv1 = dinv_ref[1, :]
    agg0 = (z0_ref[...] + y0_ref[...]) * dinv0[:, None]
    agg1 = (z1_ref[...] + y1_ref[...]) * dinv1[:, None]
    h0 = jnp.maximum(
        jnp.dot(agg0, w_ref[0], preferred_element_type=jnp.float32) + b_ref[0],
        0.0)
    h1 = jnp.maximum(
        jnp.dot(agg1, w_ref[1], preferred_element_type=jnp.float32) + b_ref[1],
        0.0)
    o0_ref[...] = h0 * dinv0[:, None]
    o1_ref[...] = h1 * dinv1[:, None]


def _mid(z0, z1, y0, y1, dinv, w1s, b1s):
    return pl.pallas_call(
        _mid_body,
        grid=(NP // BLK,),
        in_specs=[
            pl.BlockSpec((BLK, D), lambda i: (i, 0)),
            pl.BlockSpec((BLK, D), lambda i: (i, 0)),
            pl.BlockSpec((BLK, D), lambda i: (i, 0)),
            pl.BlockSpec((BLK, D), lambda i: (i, 0)),
            pl.BlockSpec((2, BLK), lambda i: (0, i)),
            pl.BlockSpec((2, D, D), lambda i: (0, 0, 0)),
            pl.BlockSpec((2, D), lambda i: (0, 0)),
        ],
        out_specs=[
            pl.BlockSpec((BLK, D), lambda i: (i, 0)),
            pl.BlockSpec((BLK, D), lambda i: (i, 0)),
        ],
        out_shape=[
            jax.ShapeDtypeStruct((NP, D), jnp.float32),
            jax.ShapeDtypeStruct((NP, D), jnp.float32),
        ],
    )(z0, z1, y0, y1, dinv, w1s, b1s)


def _fin_body(z0_ref, z1_ref, y0_ref, y1_ref, dinv_ref, w2_ref, b2_ref,
              wc_ref, bc_ref, out_ref, oo_ref, dd_ref):
    dinv0 = dinv_ref[0, :]
    dinv1 = dinv_ref[1, :]
    agg0 = (z0_ref[...] + y0_ref[...]) * dinv0[:, None]
    agg1 = (z1_ref[...] + y1_ref[...]) * dinv1[:, None]
    oo = jnp.dot(agg0, w2_ref[0], preferred_element_type=jnp.float32) + b2_ref[0]
    dd = jnp.dot(agg1, w2_ref[1], preferred_element_type=jnp.float32) + b2_ref[1]
    oo_ref[...] = oo
    dd_ref[...] = dd
    out = (jnp.dot(oo, wc_ref[0], preferred_element_type=jnp.float32)
           + jnp.dot(dd, wc_ref[1], preferred_element_type=jnp.float32)
           + bc_ref[...])
    out_ref[...] = jnp.maximum(out, 0.0)


def _fin(z0, z1, y0, y1, dinv, w2s, b2s, wcs, bc):
    return pl.pallas_call(
        _fin_body,
        grid=(NP // BLK,),
        in_specs=[
            pl.BlockSpec((BLK, D), lambda i: (i, 0)),
            pl.BlockSpec((BLK, D), lambda i: (i, 0)),
            pl.BlockSpec((BLK, D), lambda i: (i, 0)),
            pl.BlockSpec((BLK, D), lambda i: (i, 0)),
            pl.BlockSpec((2, BLK), lambda i: (0, i)),
            pl.BlockSpec((2, D, D), lambda i: (0, 0, 0)),
            pl.BlockSpec((2, D), lambda i: (0, 0)),
            pl.BlockSpec((2, D, D), lambda i: (0, 0, 0)),
            pl.BlockSpec((D,), lambda i: (0,)),
        ],
        out_specs=[
            pl.BlockSpec((BLK, D), lambda i: (i, 0)),
            pl.BlockSpec((BLK, D), lambda i: (i, 0)),
            pl.BlockSpec((BLK, D), lambda i: (i, 0)),
        ],
        out_shape=[
            jax.ShapeDtypeStruct((NP, D), jnp.float32),
            jax.ShapeDtypeStruct((NP, D), jnp.float32),
            jax.ShapeDtypeStruct((NP, D), jnp.float32),
        ],
    )(z0, z1, y0, y1, dinv, w2s, b2s, wcs, bc)


def _pad_edges(ei):
    src = jnp.concatenate(
        [ei[0], jnp.full((EP - E,), N, jnp.int32)]).reshape(NS, K, CH)
    dst = jnp.concatenate(
        [ei[1], jnp.full((EP - E,), N, jnp.int32)]).reshape(NS, K, CH)
    return src, dst


def kernel(x, original_edge_index, dg_edge_index,
           Wo1, bo1, Wo2, bo2, Wd1, bd1, Wd2, bd2, Wc, bc):
    src0, dst0 = _pad_edges(original_edge_index)
    src1, dst1 = _pad_edges(dg_edge_index)
    dst0f = dst0.reshape(NS, EPT)
    dst1f = dst1.reshape(NS, EPT)
    xp = jnp.zeros((NP, D), jnp.float32).at[:N].set(x)
    zr = jnp.zeros((NP, D), jnp.float32)

    degp0, degp1 = _deg_kernel(dst0f, dst1f)
    dinv, y0, y1 = _prep(degp0, degp1, xp)
    z0, z1 = _prop_kernel(y0, y1, src0, dst0, src1, dst1, zr)
    y20, y21 = _mid(z0, z1, y0, y1, dinv,
                    jnp.stack([Wo1, Wd1]), jnp.stack([bo1, bd1]))
    z20, z21 = _prop_kernel(y20, y21, src0, dst0, src1, dst1, zr)
    out, oo, dd = _fin(z20, z21, y20, y21, dinv,
                       jnp.stack([Wo2, Wd2]), jnp.stack([bo2, bd2]),
                       Wc.reshape(2, D, D), bc)

    out = out[:N]
    oo = oo[:N]
    dd = dd[:N]
    concat = jnp.concatenate([oo, dd], axis=-1)
    return (out, oo, dd, concat)


# speedup vs baseline: 1.0044x; 1.0044x over previous
import functools

import jax
import jax.numpy as jnp
from jax import lax
from jax.experimental import pallas as pl
from jax.experimental.pallas import tpu as pltpu
from jax.experimental.pallas import tpu_sc as plsc

N = 10000
D = 128
E = 320000

NC = 2
NS = 16
CH = 128
K = 160
KBI = 40
NB = K // KBI
EPT = K * CH
EP = EPT * NS
NP = 10240
ROWS_PT = NP // NS

_mesh = plsc.VectorSubcoreMesh(core_axis_name="c", subcore_axis_name="s")


@functools.partial(
    pl.kernel,
    out_type=[
        jax.ShapeDtypeStruct((NS, NP), jnp.float32),
        jax.ShapeDtypeStruct((NS, NP), jnp.float32),
    ],
    mesh=_mesh,
    scratch_types=[
        pltpu.VMEM((EPT,), jnp.int32),
        pltpu.VMEM((NP,), jnp.float32),
    ],
    compiler_params=pltpu.CompilerParams(needs_layout_passes=False),
)
def _deg_kernel(dst0, dst1, degp0, degp1, dst_v, deg_v):
    cid = lax.axis_index("c")
    sid = lax.axis_index("s")
    zeros16 = jnp.zeros((16,), jnp.float32)
    ones16 = jnp.ones((16,), jnp.float32)

    @pl.loop(0, NP // 16)
    def _(i):
        deg_v[pl.ds(i * 16, 16)] = zeros16

    def work(dstb, degpb):
        pltpu.sync_copy(dstb.at[sid], dst_v)

        @pl.loop(0, EPT // 16)
        def _(i):
            idx = dst_v[pl.ds(i * 16, 16)]
            plsc.addupdate_scatter(deg_v, [idx], ones16)

        pltpu.sync_copy(deg_v, degpb.at[sid])

    @pl.when(cid == 0)
    def _():
        work(dst0, degp0)

    @pl.when(cid == 1)
    def _():
        work(dst1, degp1)


@functools.partial(
    pl.kernel,
    out_type=[
        jax.ShapeDtypeStruct((NP, D), jnp.float32),
        jax.ShapeDtypeStruct((NP, D), jnp.float32),
    ],
    mesh=_mesh,
    scratch_types=[
        pltpu.VMEM((KBI, CH), jnp.int32),
        pltpu.VMEM((KBI, CH), jnp.int32),
        pltpu.VMEM((CH, D), jnp.float32),
        pltpu.VMEM((CH, D), jnp.float32),
        pltpu.VMEM_SHARED((NP, D), jnp.float32),
        pltpu.SemaphoreType.DMA,
        pltpu.SemaphoreType.DMA,
    ],
    compiler_params=pltpu.CompilerParams(needs_layout_passes=False),
)
def _prop_kernel(y0, y1, src0, dst0, src1, dst1, zr, z0, z1,
                 src_v, dst_v, rows0_v, rows1_v, agg, sem0, sem1):
    cid = lax.axis_index("c")
    sid = lax.axis_index("s")

    pltpu.sync_copy(zr.at[pl.ds(sid * ROWS_PT, ROWS_PT)],
                    agg.at[pl.ds(sid * ROWS_PT, ROWS_PT)])
    plsc.subcore_barrier()

    def work(yb, srcb, dstb, zb):
        @pl.loop(0, NB)
        def _(i):
            pltpu.sync_copy(srcb.at[sid, pl.ds(i * KBI, KBI)], src_v)
            pltpu.sync_copy(dstb.at[sid, pl.ds(i * KBI, KBI)], dst_v)
            g0 = pltpu.async_copy(yb.at[src_v.at[0]], rows0_v, sem0)
            g1 = pltpu.async_copy(yb.at[src_v.at[1]], rows1_v, sem1)

            @pl.loop(0, KBI, step=2)
            def _(j):
                g0.wait()
                pltpu.sync_copy(rows0_v, agg.at[dst_v.at[j]], add=True)

                @pl.when(j + 2 < KBI)
                def _():
                    pltpu.async_copy(yb.at[src_v.at[j + 2]], rows0_v, sem0)

                g1.wait()
                pltpu.sync_copy(rows1_v, agg.at[dst_v.at[j + 1]], add=True)

                @pl.when(j + 3 < KBI)
                def _():
                    pltpu.async_copy(yb.at[src_v.at[j + 3]], rows1_v, sem1)

    @pl.when(cid == 0)
    def _():
        work(y0, src0, dst0, z0)

    @pl.when(cid == 1)
    def _():
        work(y1, src1, dst1, z1)

    plsc.subcore_barrier()

    @pl.when(cid == 0)
    def _():
        pltpu.sync_copy(agg.at[pl.ds(sid * ROWS_PT, ROWS_PT)],
                        z0.at[pl.ds(sid * ROWS_PT, ROWS_PT)])

    @pl.when(cid == 1)
    def _():
        pltpu.sync_copy(agg.at[pl.ds(sid * ROWS_PT, ROWS_PT)],
                        z1.at[pl.ds(sid * ROWS_PT, ROWS_PT)])


BLK = 512


def _prep_body(degp0_ref, degp1_ref, xp_ref, dinv_ref, y0_ref, y1_ref):
    deg0 = jnp.sum(degp0_ref[...], axis=0) + 1.0
    deg1 = jnp.sum(degp1_ref[...], axis=0) + 1.0
    dinv0 = lax.rsqrt(jnp.maximum(deg0, 1.0))
    dinv1 = lax.rsqrt(jnp.maximum(deg1, 1.0))
    dinv_ref[0, :] = dinv0
    dinv_ref[1, :] = dinv1
    x = xp_ref[...]
    y0_ref[...] = x * dinv0[:, None]
    y1_ref[...] = x * dinv1[:, None]


def _prep(degp0, degp1, xp):
    return pl.pallas_call(
        _prep_body,
        grid=(NP // BLK,),
        in_specs=[
            pl.BlockSpec((NS, BLK), lambda i: (0, i)),
            pl.BlockSpec((NS, BLK), lambda i: (0, i)),
            pl.BlockSpec((BLK, D), lambda i: (i, 0)),
        ],
        out_specs=[
            pl.BlockSpec((2, BLK), lambda i: (0, i)),
            pl.BlockSpec((BLK, D), lambda i: (i, 0)),
            pl.BlockSpec((BLK, D), lambda i: (i, 0)),
        ],
        out_shape=[
            jax.ShapeDtypeStruct((2, NP), jnp.float32),
            jax.ShapeDtypeStruct((NP, D), jnp.float32),
            jax.ShapeDtypeStruct((NP, D), jnp.float32),
        ],
    )(degp0, degp1, xp)


def _mid_body(z0_ref, z1_ref, y0_ref, y1_ref, dinv_ref, w_ref, b_ref,
              o0_ref, o1_ref):
    dinv0 = dinv_ref[0, :]
    dinv1 = dinv_ref[1, :]
    agg0 = (z0_ref[...] + y0_ref[...]) * dinv0[:, None]
    agg1 = (z1_ref[...] + y1_ref[...]) * dinv1[:, None]
    h0 = jnp.maximum(
        jnp.dot(agg0, w_ref[0], preferred_element_type=jnp.float32) + b_ref[0],
        0.0)
    h1 = jnp.maximum(
        jnp.dot(agg1, w_ref[1], preferred_element_type=jnp.float32) + b_ref[1],
        0.0)
    o0_ref[...] = h0 * dinv0[:, None]
    o1_ref[...] = h1 * dinv1[:, None]


def _mid(z0, z1, y0, y1, dinv, w1s, b1s):
    return pl.pallas_call(
        _mid_body,
        grid=(NP // BLK,),
        in_specs=[
            pl.BlockSpec((BLK, D), lambda i: (i, 0)),
            pl.BlockSpec((BLK, D), lambda i: (i, 0)),
            pl.BlockSpec((BLK, D), lambda i: (i, 0)),
            pl.BlockSpec((BLK, D), lambda i: (i, 0)),
            pl.BlockSpec((2, BLK), lambda i: (0, i)),
            pl.BlockSpec((2, D, D), lambda i: (0, 0, 0)),
            pl.BlockSpec((2, D), lambda i: (0, 0)),
        ],
        out_specs=[
            pl.BlockSpec((BLK, D), lambda i: (i, 0)),
            pl.BlockSpec((BLK, D), lambda i: (i, 0)),
        ],
        out_shape=[
            jax.ShapeDtypeStruct((NP, D), jnp.float32),
            jax.ShapeDtypeStruct((NP, D), jnp.float32),
        ],
    )(z0, z1, y0, y1, dinv, w1s, b1s)


def _fin_body(z0_ref, z1_ref, y0_ref, y1_ref, dinv_ref, w2_ref, b2_ref,
              wc_ref, bc_ref, out_ref, oo_ref, dd_ref):
    dinv0 = dinv_ref[0, :]
    dinv1 = dinv_ref[1, :]
    agg0 = (z0_ref[...] + y0_ref[...]) * dinv0[:, None]
    agg1 = (z1_ref[...] + y1_ref[...]) * dinv1[:, None]
    oo = jnp.dot(agg0, w2_ref[0], preferred_element_type=jnp.float32) + b2_ref[0]
    dd = jnp.dot(agg1, w2_ref[1], preferred_element_type=jnp.float32) + b2_ref[1]
    oo_ref[...] = oo
    dd_ref[...] = dd
    out = (jnp.dot(oo, wc_ref[0], preferred_element_type=jnp.float32)
           + jnp.dot(dd, wc_ref[1], preferred_element_type=jnp.float32)
           + bc_ref[...])
    out_ref[...] = jnp.maximum(out, 0.0)


def _fin(z0, z1, y0, y1, dinv, w2s, b2s, wcs, bc):
    return pl.pallas_call(
        _fin_body,
        grid=(NP // BLK,),
        in_specs=[
            pl.BlockSpec((BLK, D), lambda i: (i, 0)),
            pl.BlockSpec((BLK, D), lambda i: (i, 0)),
            pl.BlockSpec((BLK, D), lambda i: (i, 0)),
            pl.BlockSpec((BLK, D), lambda i: (i, 0)),
            pl.BlockSpec((2, BLK), lambda i: (0, i)),
            pl.BlockSpec((2, D, D), lambda i: (0, 0, 0)),
            pl.BlockSpec((2, D), lambda i: (0, 0)),
            pl.BlockSpec((2, D, D), lambda i: (0, 0, 0)),
            pl.BlockSpec((D,), lambda i: (0,)),
        ],
        out_specs=[
            pl.BlockSpec((BLK, D), lambda i: (i, 0)),
            pl.BlockSpec((BLK, D), lambda i: (i, 0)),
            pl.BlockSpec((BLK, D), lambda i: (i, 0)),
        ],
        out_shape=[
            jax.ShapeDtypeStruct((NP, D), jnp.float32),
            jax.ShapeDtypeStruct((NP, D), jnp.float32),
            jax.ShapeDtypeStruct((NP, D), jnp.float32),
        ],
    )(z0, z1, y0, y1, dinv, w2s, b2s, wcs, bc)


def _pad_edges(ei):
    src = jnp.concatenate(
        [ei[0], jnp.full((EP - E,), N, jnp.int32)]).reshape(NS, K, CH)
    dst = jnp.concatenate(
        [ei[1], jnp.full((EP - E,), N, jnp.int32)]).reshape(NS, K, CH)
    return src, dst


def kernel(x, original_edge_index, dg_edge_index,
           Wo1, bo1, Wo2, bo2, Wd1, bd1, Wd2, bd2, Wc, bc):
    src0, dst0 = _pad_edges(original_edge_index)
    src1, dst1 = _pad_edges(dg_edge_index)
    dst0f = dst0.reshape(NS, EPT)
    dst1f = dst1.reshape(NS, EPT)
    xp = jnp.zeros((NP, D), jnp.float32).at[:N].set(x)
    zr = jnp.zeros((NP, D), jnp.float32)

    degp0, degp1 = _deg_kernel(dst0f, dst1f)
    dinv, y0, y1 = _prep(degp0, degp1, xp)
    z0, z1 = _prop_kernel(y0, y1, src0, dst0, src1, dst1, zr)
    y20, y21 = _mid(z0, z1, y0, y1, dinv,
                    jnp.stack([Wo1, Wd1]), jnp.stack([bo1, bd1]))
    z20, z21 = _prop_kernel(y20, y21, src0, dst0, src1, dst1, zr)
    out, oo, dd = _fin(z20, z21, y20, y21, dinv,
                       jnp.stack([Wo2, Wd2]), jnp.stack([bo2, bd2]),
                       Wc.reshape(2, D, D), bc)

    out = out[:N]
    oo = oo[:N]
    dd = dd[:N]
    concat = jnp.concatenate([oo, dd], axis=-1)
    return (out, oo, dd, concat)
